# baseline (speedup 1.0000x reference)
import jax
import jax.numpy as jnp
from jax import lax
from jax.experimental import pallas as pl
from jax.experimental.pallas import tpu as pltpu
from jax.experimental.pallas import tpu_sc as plsc

NUM_POINTS = 4096
DIMS = 128
BATCH = 512

_NC = 2
_NS = 16
_NW = _NC * _NS
_RPW = BATCH // _NW
_LANES = 16
_NQ = 2
_QR = _RPW // _NQ


def _sc_gather_body(idx_hbm, embeds_hbm, graph_hbm,
                    e_out_hbm, g_out_hbm,
                    idx_all_v, emb_v, rows_v, gsel_v,
                    sem_e, sem_g, sem_o):
  wid = lax.axis_index("s") * _NC + lax.axis_index("c")
  base = wid * _RPW

  pltpu.sync_copy(idx_hbm, idx_all_v)

  gq = [pltpu.async_copy(graph_hbm.at[idx_all_v.at[pl.ds(base + q * _QR, _QR)]],
                         rows_v.at[q], sem_g[q])
        for q in range(_NQ)]
  cp_e = pltpu.async_copy(embeds_hbm.at[idx_all_v.at[pl.ds(base, _RPW)]],
                          emb_v, sem_e)

  out_cps = []
  for q in range(_NQ):
    gq[q].wait()

    def chunk(c, carry, q=q):
      cols = idx_all_v[pl.ds(c * _LANES, _LANES)]
      for r in range(_QR):
        rvec = jnp.full((_LANES,), r, dtype=jnp.int32)
        vals = plsc.load_gather(rows_v.at[q], [rvec, cols])
        gsel_v[pl.ds((q * _QR + r) * BATCH + c * _LANES, _LANES)] = vals
      return carry

    lax.fori_loop(0, BATCH // _LANES, chunk, 0)
    out_cps.append(pltpu.async_copy(
        gsel_v.at[pl.ds(q * _QR * BATCH, _QR * BATCH)],
        g_out_hbm.at[pl.ds((base + q * _QR) * BATCH, _QR * BATCH)],
        sem_o))

  cp_e.wait()
  pltpu.sync_copy(emb_v, e_out_hbm.at[pl.ds(base, _RPW)])
  for cp in out_cps:
    cp.wait()


def _sc_gather(idx, embeds, graph):
  mesh = plsc.VectorSubcoreMesh(core_axis_name="c", subcore_axis_name="s")
  fn = pl.kernel(
      _sc_gather_body,
      out_type=(
          jax.ShapeDtypeStruct((BATCH, DIMS), jnp.float32),
          jax.ShapeDtypeStruct((BATCH * BATCH,), jnp.float32),
      ),
      mesh=mesh,
      scratch_types=[
          pltpu.VMEM((BATCH,), jnp.int32),
          pltpu.VMEM((_RPW, DIMS), jnp.float32),
          pltpu.VMEM((_NQ, _QR, NUM_POINTS), jnp.float32),
          pltpu.VMEM((_RPW * BATCH,), jnp.float32),
          pltpu.SemaphoreType.DMA,
          [pltpu.SemaphoreType.DMA] * _NQ,
          pltpu.SemaphoreType.DMA,
      ],
      compiler_params=pltpu.CompilerParams(needs_layout_passes=False),
  )
  return fn(idx, embeds, graph)


def _tc_loss_body(e_ref, g_ref, out_ref):
  e = e_ref[...]
  g = g_ref[...]
  e2 = e * e
  n_col = jnp.sum(e2, axis=1, keepdims=True)
  ones = jnp.ones((1, DIMS), dtype=jnp.float32)
  n_row = lax.dot_general(ones, e2, (((1,), (1,)), ((), ())),
                          preferred_element_type=jnp.float32,
                          precision=lax.Precision.HIGHEST)
  gram = lax.dot_general(e, e, (((1,), (1,)), ((), ())),
                         preferred_element_type=jnp.float32,
                         precision=lax.Precision.HIGHEST)
  d2 = jnp.maximum(n_col + n_row - 2.0 * gram, 0.0) + 1e-12
  loss = jnp.abs(d2 / (g * g) - 1.0)
  row = lax.broadcasted_iota(jnp.int32, (BATCH, BATCH), 0)
  col = lax.broadcasted_iota(jnp.int32, (BATCH, BATCH), 1)
  loss = jnp.where(col > row, loss, 0.0)
  out_ref[0, 0] = jnp.sum(loss)


def _tc_loss(e_rows, g_sub):
  return pl.pallas_call(
      _tc_loss_body,
      out_shape=jax.ShapeDtypeStruct((1, 1), jnp.float32),
      out_specs=pl.BlockSpec(memory_space=pltpu.SMEM),
  )(e_rows, g_sub)


def kernel(input_index, embeds, graph):
  idx = input_index.astype(jnp.int32)
  e_rows, g_flat = _sc_gather(idx, embeds, graph)
  out = _tc_loss(e_rows, g_flat.reshape(BATCH, BATCH))
  return out[0, 0]
